# baseline (speedup 1.0000x reference)
import functools

import jax
import jax.numpy as jnp
from jax import lax
from jax.experimental import pallas as pl
from jax.experimental.pallas import tpu as pltpu
from jax.experimental.pallas import tpu_sc as plsc

NUM_CODES = 8192
CODE_DIM = 64
B = 18432

TB = 512
CK = 4096
_BIGF = 3.0e38


NBLK = 2
BS = NUM_CODES // NBLK


def _argmin_body(zn_ref, z_ref, cbn_ref, cbt_ref, idx_ref):
    z = z_ref[...]
    zn = zn_ref[...]

    iota_f = lax.broadcasted_iota(jnp.int32, (TB, CK), 1).astype(jnp.float32)

    def half_argmin(blk):
        run_min = None
        run_idx = None
        for c in range(BS // CK):
            base = blk * BS + c * CK
            cbt = cbt_ref[:, pl.ds(base, CK)]
            cn = cbn_ref[:, pl.ds(base, CK)]
            dots2 = lax.dot_general(z, cbt, (((1,), (0,)), ((), ())),
                                    preferred_element_type=jnp.float32)
            d = (zn + cn) - dots2
            cmin = jnp.min(d, axis=1, keepdims=True)
            cidx = jnp.min(jnp.where(d == cmin, iota_f, _BIGF), axis=1,
                           keepdims=True) + float(base)
            if run_min is None:
                run_min, run_idx = cmin, cidx
            else:
                better = cmin < run_min
                run_idx = jnp.where(better, cidx, run_idx)
                run_min = jnp.minimum(cmin, run_min)
        return run_min, run_idx

    m0, i0 = half_argmin(0)
    m1, i1 = half_argmin(1)
    m0q = m0.astype(jnp.bfloat16).astype(jnp.float32)
    win = m1 < m0q
    idx_ref[...] = jnp.where(win, i1, i0).astype(jnp.int32)


def _argmin_call(z_norm, z_e, cb_norm, cb_t):
    grid = (B // TB,)
    return pl.pallas_call(
        _argmin_body,
        grid=grid,
        in_specs=[
            pl.BlockSpec((TB, 1), lambda i: (i, 0)),
            pl.BlockSpec((TB, CODE_DIM), lambda i: (i, 0)),
            pl.BlockSpec((1, NUM_CODES), lambda i: (0, 0)),
            pl.BlockSpec((CODE_DIM, NUM_CODES), lambda i: (0, 0)),
        ],
        out_specs=pl.BlockSpec((TB, 1), lambda i: (i, 0)),
        out_shape=jax.ShapeDtypeStruct((B, 1), jnp.int32),
    )(z_norm, z_e, cb_norm, cb_t)


GD = 128


def _make_sc_gather():
    info = plsc.get_sparse_core_info()
    nw = info.num_cores * info.num_subcores
    b_per_w = B // nw
    mesh = plsc.VectorSubcoreMesh(core_axis_name="c", subcore_axis_name="s")

    @functools.partial(
        pl.kernel, mesh=mesh,
        out_type=jax.ShapeDtypeStruct((B, GD), jnp.float32),
        scratch_types=[
            pltpu.VMEM((b_per_w,), jnp.int32),
            pltpu.VMEM((b_per_w, GD), jnp.float32),
            pltpu.SemaphoreType.DMA,
        ],
    )
    def sc_gather(table_hbm, idx_hbm, out_hbm, idx_v, rows_v, sem):
        wid = lax.axis_index("s") * info.num_cores + lax.axis_index("c")
        base = wid * b_per_w
        pltpu.sync_copy(idx_hbm.at[pl.ds(base, b_per_w)], idx_v)
        pltpu.async_copy(table_hbm.at[idx_v], rows_v, sem).wait()
        pltpu.sync_copy(rows_v, out_hbm.at[pl.ds(base, b_per_w)])

    return sc_gather


def kernel(z_e, codebook):
    z_norm = jnp.sum(z_e * z_e, axis=1, keepdims=True)
    cb_norm = jnp.sum(codebook * codebook, axis=1, keepdims=True)
    z_bf = (z_e + z_e).astype(jnp.bfloat16)
    cbt_bf = codebook.T.astype(jnp.bfloat16)
    idx2d = _argmin_call(z_norm, z_bf, cb_norm.T, cbt_bf)
    indices = idx2d[:, 0]
    cb_pad = jnp.pad(codebook, ((0, 0), (0, GD - CODE_DIM)))
    z_q = _make_sc_gather()(cb_pad, indices)[:, :CODE_DIM]
    return (z_q, indices)


# speedup vs baseline: 1.1216x; 1.1216x over previous
import functools

import jax
import jax.numpy as jnp
from jax import lax
from jax.experimental import pallas as pl
from jax.experimental.pallas import tpu as pltpu
from jax.experimental.pallas import tpu_sc as plsc

NUM_CODES = 8192
CODE_DIM = 64
B = 18432

TB = 512
CK = 4096
_BIGF = 3.0e38


NBLK = 2
BS = NUM_CODES // NBLK


def _argmin_body(zn_ref, z_ref, cbn_ref, cbt_ref, idx_ref):
    z = z_ref[...].astype(jnp.bfloat16)
    zn = zn_ref[...]

    iota_f = lax.broadcasted_iota(jnp.int32, (TB, CK), 1).astype(jnp.float32)

    def half_argmin(blk):
        run_min = None
        run_idx = None
        for c in range(BS // CK):
            base = blk * BS + c * CK
            cbt = cbt_ref[:, pl.ds(base, CK)]
            cn = cbn_ref[:, pl.ds(base, CK)]
            dots = lax.dot_general(z, cbt, (((1,), (0,)), ((), ())),
                                   preferred_element_type=jnp.float32)
            d = (zn + cn) - 2.0 * dots
            cmin = jnp.min(d, axis=1, keepdims=True)
            cidx = jnp.min(jnp.where(d == cmin, iota_f, _BIGF), axis=1,
                           keepdims=True) + float(base)
            if run_min is None:
                run_min, run_idx = cmin, cidx
            else:
                better = cmin < run_min
                run_idx = jnp.where(better, cidx, run_idx)
                run_min = jnp.minimum(cmin, run_min)
        return run_min, run_idx

    m0, i0 = half_argmin(0)
    m1, i1 = half_argmin(1)
    m0q = m0.astype(jnp.bfloat16).astype(jnp.float32)
    win = m1 < m0q
    idx_ref[...] = jnp.where(win, i1, i0).astype(jnp.int32)


def _argmin_call(z_norm, z_e, cb_norm, cb_t):
    grid = (B // TB,)
    return pl.pallas_call(
        _argmin_body,
        grid=grid,
        in_specs=[
            pl.BlockSpec((TB, 1), lambda i: (i, 0)),
            pl.BlockSpec((TB, CODE_DIM), lambda i: (i, 0)),
            pl.BlockSpec((1, NUM_CODES), lambda i: (0, 0)),
            pl.BlockSpec((CODE_DIM, NUM_CODES), lambda i: (0, 0)),
        ],
        out_specs=pl.BlockSpec((TB, 1), lambda i: (i, 0)),
        out_shape=jax.ShapeDtypeStruct((B, 1), jnp.int32),
    )(z_norm, z_e, cb_norm, cb_t)


GD = 128


def _make_sc_gather():
    info = plsc.get_sparse_core_info()
    nw = info.num_cores * info.num_subcores
    b_per_w = B // nw
    mesh = plsc.VectorSubcoreMesh(core_axis_name="c", subcore_axis_name="s")

    @functools.partial(
        pl.kernel, mesh=mesh,
        out_type=jax.ShapeDtypeStruct((B, GD), jnp.float32),
        scratch_types=[
            pltpu.VMEM((b_per_w,), jnp.int32),
            pltpu.VMEM((b_per_w, GD), jnp.float32),
            pltpu.SemaphoreType.DMA,
        ],
    )
    def sc_gather(table_hbm, idx_hbm, out_hbm, idx_v, rows_v, sem):
        wid = lax.axis_index("s") * info.num_cores + lax.axis_index("c")
        base = wid * b_per_w
        pltpu.sync_copy(idx_hbm.at[pl.ds(base, b_per_w)], idx_v)
        pltpu.async_copy(table_hbm.at[idx_v], rows_v, sem).wait()
        pltpu.sync_copy(rows_v, out_hbm.at[pl.ds(base, b_per_w)])

    return sc_gather


def kernel(z_e, codebook):
    z_norm = jnp.sum(z_e * z_e, axis=1, keepdims=True)
    cb_norm = jnp.sum(codebook * codebook, axis=1, keepdims=True)
    cbt_bf = codebook.T.astype(jnp.bfloat16)
    idx2d = _argmin_call(z_norm, z_e, cb_norm.T, cbt_bf)
    indices = idx2d[:, 0]
    cb_pad = jnp.pad(codebook, ((0, 0), (0, GD - CODE_DIM)))
    z_q = _make_sc_gather()(cb_pad, indices)[:, :CODE_DIM]
    return (z_q, indices)
